# baseline (speedup 1.0000x reference)
import functools

import jax
import jax.numpy as jnp
from jax import lax
from jax.experimental import pallas as pl
from jax.experimental.pallas import tpu as pltpu
from jax.experimental.pallas import tpu_sc as plsc

B, N, C, K, BINS, M, O = 4, 10000, 32, 32, 8, 1, 32
PTS = B * N
NC, NS = 2, 16
NW = NC * NS
PAD_PTS = 40960
PW = PAD_PTS // NW
BLK = 4
ROWS = BLK * K
BLOCKS = PW // BLK
HALF = PW // 2
HBLOCKS = BLOCKS // 2
IDX_ROWS = PAD_PTS * K // ROWS


NBUF = 4


def _sc_dw_kernel(table, idx, filt, cnt, dk, rcp, out,
                  dk_v, idx_v, filt_v, cnt_v, rcp_v, rows_v, out_v, *sems):
    wid = lax.axis_index("s") * NC + lax.axis_index("c")
    blk_base = wid * BLOCKS
    pt_base = wid * PW

    pltpu.sync_copy(dk.at[:, :], dk_v)
    pltpu.sync_copy(rcp.at[:], rcp_v)
    pltpu.sync_copy(idx.at[pl.ds(blk_base, BLOCKS)], idx_v)
    pltpu.sync_copy(filt.at[pl.ds(blk_base, BLOCKS)], filt_v)
    pltpu.sync_copy(cnt.at[pl.ds(pt_base, PW)], cnt_v.at[pl.ds(0, PW)])

    def gather(g, par, sem):
        return pltpu.async_copy(table.at[idx_v.at[g]], rows_v.at[par], sem)

    def compute_block(g, par):
        lp = (g % HBLOCKS) * BLK
        cl = cnt_v[pl.ds(g * BLK, 16)]
        ci = jnp.clip(jnp.maximum(cl, 1) - 1, 0, K - 1)
        invv = plsc.load_gather(rcp_v, [ci])
        for p in range(BLK):
            fv0 = filt_v[g, pl.ds(p * K, 16)]
            fv1 = filt_v[g, pl.ds(p * K + 16, 16)]
            acc0 = jnp.zeros((16,), jnp.float32)
            acc1 = jnp.zeros((16,), jnp.float32)
            for k in range(K):
                r = p * K + k
                bin_ = (fv0 if k < 16 else fv1)[k % 16]
                acc0 = acc0 + rows_v[par, r, pl.ds(0, 16)] * dk_v[bin_, pl.ds(0, 16)]
                acc1 = acc1 + rows_v[par, r, pl.ds(16, 16)] * dk_v[bin_, pl.ds(16, 16)]
            inv = invv[p]
            out_v[lp + p, pl.ds(0, 16)] = acc0 * inv
            out_v[lp + p, pl.ds(16, 16)] = acc1 * inv

    def half(h, _):
        h0 = h * HBLOCKS
        for j in range(NBUF - 1):
            gather(h0 + j, j, sems[j])

        def body(bb, _):
            for j in range(NBUF):
                b = h0 + NBUF * bb + j
                nj = (j + NBUF - 1) % NBUF

                @pl.when(NBUF * bb + j + NBUF - 1 < HBLOCKS)
                def _():
                    gather(b + NBUF - 1, nj, sems[nj])

                pltpu.make_async_copy(table.at[idx_v.at[b]], rows_v.at[j],
                                      sems[j]).wait()
                compute_block(b, j)
            return 0

        lax.fori_loop(0, HBLOCKS // NBUF, body, 0)
        pltpu.sync_copy(out_v, out.at[pl.ds(pt_base + h * HALF, HALF)])
        return 0

    lax.fori_loop(0, 2, half, 0)


def _make_sc_call():
    mesh = plsc.VectorSubcoreMesh(core_axis_name="c", subcore_axis_name="s",
                                  num_cores=NC, num_subcores=NS)
    return pl.kernel(
        _sc_dw_kernel,
        out_type=jax.ShapeDtypeStruct((PAD_PTS, C), jnp.float32),
        mesh=mesh,
        compiler_params=pltpu.CompilerParams(use_tc_tiling_on_sc=False,
                                             needs_layout_passes=False),
        scratch_types=[
            pltpu.VMEM((BINS, C), jnp.float32),
            pltpu.VMEM((BLOCKS, ROWS), jnp.int32),
            pltpu.VMEM((BLOCKS, ROWS), jnp.int32),
            pltpu.VMEM((PW + 16,), jnp.int32),
            pltpu.VMEM((K,), jnp.float32),
            pltpu.VMEM((NBUF, ROWS, C), jnp.float32),
            pltpu.VMEM((HALF, C), jnp.float32),
        ] + [pltpu.SemaphoreType.DMA] * NBUF,
    )


PACK = 4
PROWS = PTS // PACK


def _fold4(x):
    s = x[:, 0:O] + x[:, O:2 * O] + x[:, 2 * O:3 * O] + x[:, 3 * O:4 * O]
    return s, jnp.concatenate([s, s, s, s], axis=1)


def _tc_body(dw_ref, w_ref, b_ref, g_ref, be_ref, y_ref):
    x = dw_ref[pl.ds(0, PROWS), :]
    y = jnp.dot(x, w_ref[:, :], preferred_element_type=jnp.float32) + b_ref[:, :]
    _, m = _fold4(jnp.sum(y, axis=0, keepdims=True) * (1.0 / PTS))
    d = y - m
    _, v = _fold4(jnp.sum(d * d, axis=0, keepdims=True) * (1.0 / PTS))
    scale = g_ref[:, :] / jnp.sqrt(v + 1e-5)
    y_ref[:, :] = jnp.maximum(d * scale + be_ref[:, :], 0.0)


def kernel(inputs, nn_index, nn_count, filt_index, depthwise_kernel, fc_w, fc_b, gamma, beta):
    table = inputs.reshape(PTS, C)
    offs = (jnp.arange(B, dtype=jnp.int32) * N)[:, None, None]
    idx_flat = (nn_index + offs).reshape(PTS * K)
    pad_k = jnp.zeros(((PAD_PTS - PTS) * K,), jnp.int32)
    idx2 = jnp.concatenate([idx_flat, pad_k]).reshape(IDX_ROWS, ROWS)
    filt2 = jnp.concatenate([filt_index.reshape(PTS * K), pad_k]).reshape(IDX_ROWS, ROWS)
    cnt1 = jnp.concatenate([nn_count.reshape(PTS),
                            jnp.ones((PAD_PTS - PTS,), jnp.int32)])
    dk2 = depthwise_kernel.reshape(BINS, C * M)
    rcp = 1.0 / jnp.arange(1, K + 1, dtype=jnp.float32)

    dw = _make_sc_call()(table, idx2, filt2, cnt1, dk2, rcp)

    w_bd = jnp.kron(jnp.eye(PACK, dtype=jnp.float32), fc_w)
    b_t = jnp.tile(fc_b, PACK).reshape(1, PACK * O)
    g_t = jnp.tile(gamma, PACK).reshape(1, PACK * O)
    be_t = jnp.tile(beta, PACK).reshape(1, PACK * O)
    y = pl.pallas_call(
        _tc_body,
        out_shape=jax.ShapeDtypeStruct((PROWS, PACK * O), jnp.float32),
    )(dw.reshape(PAD_PTS // PACK, PACK * C), w_bd, b_t, g_t, be_t)
    return y.reshape(B, N, O)


# speedup vs baseline: 1.1563x; 1.1563x over previous
import functools

import jax
import jax.numpy as jnp
from jax import lax
from jax.experimental import pallas as pl
from jax.experimental.pallas import tpu as pltpu
from jax.experimental.pallas import tpu_sc as plsc

B, N, C, K, BINS, M, O = 4, 10000, 32, 32, 8, 1, 32
PTS = B * N
NC, NS = 2, 16
NW = NC * NS
PAD_PTS = 40960
PW = PAD_PTS // NW
BLK = 4
ROWS = BLK * K
BLOCKS = PW // BLK
HALF = PW // 2
HBLOCKS = BLOCKS // 2
IDX_ROWS = PAD_PTS * K // ROWS


NBUF = 4
CHUNK_BLKS = 32
CHUNK_PTS = CHUNK_BLKS * BLK
NCHUNK = BLOCKS // CHUNK_BLKS


def _sc_dw_kernel(table, idx, filt, cnt, dk, rcp, out,
                  dk_v, idx_v0, idx_v1, filt_v0, filt_v1, cnt_v, rcp_v,
                  rows_v, out_v, tab_sh, sem_c0, sem_c1, *sems):
    c = lax.axis_index("c")
    s = lax.axis_index("s")
    wid = s * NC + c
    blk_base = wid * BLOCKS
    pt_base = wid * PW

    @pl.when(s == 0)
    def _():
        pltpu.sync_copy(table.at[:, :], tab_sh)

    pltpu.sync_copy(dk.at[:, :], dk_v)
    pltpu.sync_copy(rcp.at[:], rcp_v)
    pltpu.sync_copy(cnt.at[pl.ds(pt_base, PW)], cnt_v.at[pl.ds(0, PW)])
    plsc.subcore_barrier()

    idx_b = (idx_v0, idx_v1)
    filt_b = (filt_v0, filt_v1)
    csem = (sem_c0, sem_c1)

    def chunk_load(ci, par):
        base = blk_base + ci * CHUNK_BLKS
        pltpu.async_copy(idx.at[pl.ds(base, CHUNK_BLKS)], idx_b[par], csem[par])
        pltpu.async_copy(filt.at[pl.ds(base, CHUNK_BLKS)], filt_b[par], csem[par])

    def chunk_wait(par):
        pltpu.make_async_copy(idx.at[pl.ds(0, CHUNK_BLKS)], idx_b[par],
                              csem[par]).wait()
        pltpu.make_async_copy(filt.at[pl.ds(0, CHUNK_BLKS)], filt_b[par],
                              csem[par]).wait()

    def gather(idxb, b, j, sem):
        return pltpu.async_copy(tab_sh.at[idxb.at[b]], rows_v.at[j], sem)

    def compute_block(ci, b, j, filtb):
        lp = b * BLK
        cl = cnt_v[pl.ds(ci * CHUNK_PTS + b * BLK, 16)]
        cidx = jnp.clip(jnp.maximum(cl, 1) - 1, 0, K - 1)
        invv = plsc.load_gather(rcp_v, [cidx])
        for p in range(BLK):
            fv0 = filtb[b, pl.ds(p * K, 16)]
            fv1 = filtb[b, pl.ds(p * K + 16, 16)]
            acc0 = jnp.zeros((16,), jnp.float32)
            acc1 = jnp.zeros((16,), jnp.float32)
            for k in range(K):
                r = p * K + k
                bin_ = (fv0 if k < 16 else fv1)[k % 16]
                acc0 = acc0 + rows_v[j, r, pl.ds(0, 16)] * dk_v[bin_, pl.ds(0, 16)]
                acc1 = acc1 + rows_v[j, r, pl.ds(16, 16)] * dk_v[bin_, pl.ds(16, 16)]
            inv = invv[p]
            out_v[lp + p, pl.ds(0, 16)] = acc0 * inv
            out_v[lp + p, pl.ds(16, 16)] = acc1 * inv

    def do_chunk(ci, parc):
        chunk_wait(parc)

        @pl.when(ci + 1 < NCHUNK)
        def _():
            chunk_load(ci + 1, 1 - parc)

        idxb, filtb = idx_b[parc], filt_b[parc]
        for j in range(NBUF - 1):
            gather(idxb, j, j, sems[j])

        def body(bb, _):
            for j in range(NBUF):
                b = NBUF * bb + j
                nj = (j + NBUF - 1) % NBUF

                @pl.when(b + NBUF - 1 < CHUNK_BLKS)
                def _():
                    gather(idxb, b + NBUF - 1, nj, sems[nj])

                pltpu.make_async_copy(tab_sh.at[idxb.at[b]], rows_v.at[j],
                                      sems[j]).wait()
                compute_block(ci, b, j, filtb)
            return 0

        lax.fori_loop(0, CHUNK_BLKS // NBUF, body, 0)
        pltpu.sync_copy(out_v, out.at[pl.ds(pt_base + ci * CHUNK_PTS, CHUNK_PTS)])

    chunk_load(0, 0)

    def cpair(cc, _):
        do_chunk(2 * cc, 0)
        do_chunk(2 * cc + 1, 1)
        return 0

    lax.fori_loop(0, NCHUNK // 2, cpair, 0)


def _make_sc_call():
    mesh = plsc.VectorSubcoreMesh(core_axis_name="c", subcore_axis_name="s",
                                  num_cores=NC, num_subcores=NS)
    return pl.kernel(
        _sc_dw_kernel,
        out_type=jax.ShapeDtypeStruct((PAD_PTS, C), jnp.float32),
        mesh=mesh,
        compiler_params=pltpu.CompilerParams(use_tc_tiling_on_sc=False,
                                             needs_layout_passes=False),
        scratch_types=[
            pltpu.VMEM((BINS, C), jnp.float32),
            pltpu.VMEM((CHUNK_BLKS, ROWS), jnp.int32),
            pltpu.VMEM((CHUNK_BLKS, ROWS), jnp.int32),
            pltpu.VMEM((CHUNK_BLKS, ROWS), jnp.int32),
            pltpu.VMEM((CHUNK_BLKS, ROWS), jnp.int32),
            pltpu.VMEM((PW + 16,), jnp.int32),
            pltpu.VMEM((K,), jnp.float32),
            pltpu.VMEM((NBUF, ROWS, C), jnp.float32),
            pltpu.VMEM((CHUNK_PTS, C), jnp.float32),
            pltpu.VMEM_SHARED((PTS, C), jnp.float32),
            pltpu.SemaphoreType.DMA,
            pltpu.SemaphoreType.DMA,
        ] + [pltpu.SemaphoreType.DMA] * NBUF,
    )


PACK = 4
PROWS = PTS // PACK


def _fold4(x):
    s = x[:, 0:O] + x[:, O:2 * O] + x[:, 2 * O:3 * O] + x[:, 3 * O:4 * O]
    return s, jnp.concatenate([s, s, s, s], axis=1)


def _tc_body(dw_ref, w_ref, b_ref, g_ref, be_ref, y_ref):
    x = dw_ref[pl.ds(0, PROWS), :]
    y = jnp.dot(x, w_ref[:, :], preferred_element_type=jnp.float32) + b_ref[:, :]
    _, m = _fold4(jnp.sum(y, axis=0, keepdims=True) * (1.0 / PTS))
    d = y - m
    _, v = _fold4(jnp.sum(d * d, axis=0, keepdims=True) * (1.0 / PTS))
    scale = g_ref[:, :] / jnp.sqrt(v + 1e-5)
    y_ref[:, :] = jnp.maximum(d * scale + be_ref[:, :], 0.0)


def kernel(inputs, nn_index, nn_count, filt_index, depthwise_kernel, fc_w, fc_b, gamma, beta):
    table = inputs.reshape(PTS, C)
    offs = (jnp.arange(B, dtype=jnp.int32) * N)[:, None, None]
    idx_flat = (nn_index + offs).reshape(PTS * K)
    pad_k = jnp.zeros(((PAD_PTS - PTS) * K,), jnp.int32)
    idx2 = jnp.concatenate([idx_flat, pad_k]).reshape(IDX_ROWS, ROWS)
    filt2 = jnp.concatenate([filt_index.reshape(PTS * K), pad_k]).reshape(IDX_ROWS, ROWS)
    cnt1 = jnp.concatenate([nn_count.reshape(PTS),
                            jnp.ones((PAD_PTS - PTS,), jnp.int32)])
    dk2 = depthwise_kernel.reshape(BINS, C * M)
    rcp = 1.0 / jnp.arange(1, K + 1, dtype=jnp.float32)

    dw = _make_sc_call()(table, idx2, filt2, cnt1, dk2, rcp)

    w_bd = jnp.kron(jnp.eye(PACK, dtype=jnp.float32), fc_w)
    b_t = jnp.tile(fc_b, PACK).reshape(1, PACK * O)
    g_t = jnp.tile(gamma, PACK).reshape(1, PACK * O)
    be_t = jnp.tile(beta, PACK).reshape(1, PACK * O)
    y = pl.pallas_call(
        _tc_body,
        out_shape=jax.ShapeDtypeStruct((PROWS, PACK * O), jnp.float32),
    )(dw.reshape(PAD_PTS // PACK, PACK * C), w_bd, b_t, g_t, be_t)
    return y.reshape(B, N, O)
